# baseline (speedup 1.0000x reference)
import functools

import jax
import jax.numpy as jnp
from jax import lax
from jax.experimental import pallas as pl
from jax.experimental.pallas import tpu as pltpu
from jax.experimental.pallas import tpu_sc as plsc

_NT = 288


def _dist_body(x_ref, e_ref, x2_ref, y2_ref, if_ref, k_ref, dist_ref, ind_ref):
    x = x_ref[...]
    e = e_ref[...]
    x2 = x2_ref[...]
    y2 = y2_ref[...]
    iota = if_ref[...]
    xy = lax.dot_general(x, e, (((1,), (1,)), ((), ())),
                         preferred_element_type=jnp.float32)
    sq = (x2 + y2) + (xy * -2.0)
    dist = -jnp.sqrt(jnp.maximum(sq, 0.0))
    dist_ref[...] = dist[None, None]

    v1 = jnp.max(dist, axis=1, keepdims=True)
    t = jnp.where(dist == v1, iota, jnp.inf)
    i1f = jnp.min(t, axis=1, keepdims=True)
    masked = jnp.where(t == i1f, -jnp.inf, dist)
    v2 = jnp.max(masked, axis=1, keepdims=True)
    i2f = jnp.min(jnp.where(masked == v2, iota, jnp.inf), axis=1, keepdims=True)
    indf = jnp.where(k_ref[...] == 0, i1f, i2f)
    ind_ref[...] = indf.astype(jnp.int32)


def _dist_topk(flat, emb, x2, y2, iota_row, k, nb, nn):
    n, d = flat.shape
    kc = emb.shape[0]
    tpb = nn // _NT
    return pl.pallas_call(
        _dist_body,
        grid=(n // _NT,),
        in_specs=[
            pl.BlockSpec((_NT, d), lambda i: (i, 0)),
            pl.BlockSpec((kc, d), lambda i: (0, 0)),
            pl.BlockSpec((_NT, 1), lambda i: (i, 0)),
            pl.BlockSpec((1, kc), lambda i: (0, 0)),
            pl.BlockSpec((1, kc), lambda i: (0, 0)),
            pl.BlockSpec((1, 1), lambda i: (0, 0)),
        ],
        out_specs=[
            pl.BlockSpec((1, 1, _NT, kc), lambda i: (0, i // tpb, i % tpb, 0)),
            pl.BlockSpec((_NT, 1), lambda i: (i, 0)),
        ],
        out_shape=[
            jax.ShapeDtypeStruct((1, nb, nn, kc), jnp.float32),
            jax.ShapeDtypeStruct((n, 1), jnp.int32),
        ],
        compiler_params=pltpu.CompilerParams(
            dimension_semantics=("parallel",),
        ),
    )(flat, emb, x2, y2, iota_row, k)


def _sc_gather(emb, ind):
    info = plsc.get_sparse_core_info()
    nc, ns = info.num_cores, info.num_subcores
    nw = nc * ns
    b = ind.shape[0]
    d = emb.shape[1]
    bpw = b // nw
    mesh = plsc.VectorSubcoreMesh(core_axis_name="c", subcore_axis_name="s")

    @functools.partial(
        pl.kernel,
        mesh=mesh,
        out_type=jax.ShapeDtypeStruct((b, d), jnp.float32),
        scratch_types=[
            pltpu.VMEM((bpw,), jnp.int32),
            pltpu.VMEM((bpw, d), jnp.float32),
            pltpu.SemaphoreType.DMA,
        ],
    )
    def gk(table_hbm, idx_hbm, out_hbm, idx_v, rows_v, sem):
        wid = lax.axis_index("s") * nc + lax.axis_index("c")
        base = wid * bpw
        pltpu.sync_copy(idx_hbm.at[pl.ds(base, bpw)], idx_v)
        pltpu.async_copy(table_hbm.at[idx_v], rows_v, sem).wait()
        pltpu.sync_copy(rows_v, out_hbm.at[pl.ds(base, bpw)])

    return gk(emb, ind)


def kernel(x, k, embed):
    b, n, d = x.shape
    kc = embed.shape[1]
    flat = x.reshape(b * n, d)
    emb = embed.reshape(kc, d)
    x2 = jnp.sum(flat ** 2, axis=-1)[:, None]
    y2 = jnp.sum(emb ** 2, axis=-1)[None, :]
    iota_row = jnp.arange(kc, dtype=jnp.float32)[None, :]
    k_arr = jnp.asarray(k, dtype=jnp.int32).reshape(1, 1)
    dist_out, ind2d = _dist_topk(flat, emb, x2, y2, iota_row, k_arr, b, n)
    ind = ind2d[:, 0]
    quant = _sc_gather(emb, ind)
    quantize = quant.reshape(b, n, d)
    embed_ind = ind.reshape(b, n)
    return quantize, embed_ind, dist_out


# speedup vs baseline: 1.0091x; 1.0091x over previous
import functools

import jax
import jax.numpy as jnp
from jax import lax
from jax.experimental import pallas as pl
from jax.experimental.pallas import tpu as pltpu
from jax.experimental.pallas import tpu_sc as plsc

_NT = 256


def _dist_body(x_ref, e_ref, x2_ref, y2_ref, if_ref, k_ref, dist_ref, ind_ref):
    x = x_ref[...]
    e = e_ref[...]
    x2 = x2_ref[...]
    y2 = y2_ref[...]
    iota = if_ref[...]
    xy = lax.dot_general(x, e, (((1,), (1,)), ((), ())),
                         preferred_element_type=jnp.float32)
    sq = (x2 + y2) + (xy * -2.0)
    dist = -jnp.sqrt(jnp.maximum(sq, 0.0))
    dist_ref[...] = dist

    v1 = jnp.max(dist, axis=1, keepdims=True)
    t = jnp.where(dist == v1, iota, jnp.inf)
    i1f = jnp.min(t, axis=1, keepdims=True)
    masked = jnp.where(t == i1f, -jnp.inf, dist)
    v2 = jnp.max(masked, axis=1, keepdims=True)
    i2f = jnp.min(jnp.where(masked == v2, iota, jnp.inf), axis=1, keepdims=True)
    indf = jnp.where(k_ref[...] == 0, i1f, i2f)
    ind_ref[...] = indf.astype(jnp.int32)


def _dist_topk(flat, emb, x2, y2, iota_row, k):
    n, d = flat.shape
    kc = emb.shape[0]
    return pl.pallas_call(
        _dist_body,
        grid=(n // _NT,),
        in_specs=[
            pl.BlockSpec((_NT, d), lambda i: (i, 0)),
            pl.BlockSpec((kc, d), lambda i: (0, 0)),
            pl.BlockSpec((_NT, 1), lambda i: (i, 0)),
            pl.BlockSpec((1, kc), lambda i: (0, 0)),
            pl.BlockSpec((1, kc), lambda i: (0, 0)),
            pl.BlockSpec((1, 1), lambda i: (0, 0)),
        ],
        out_specs=[
            pl.BlockSpec((_NT, kc), lambda i: (i, 0)),
            pl.BlockSpec((_NT, 1), lambda i: (i, 0)),
        ],
        out_shape=[
            jax.ShapeDtypeStruct((n, kc), jnp.float32),
            jax.ShapeDtypeStruct((n, 1), jnp.int32),
        ],
        compiler_params=pltpu.CompilerParams(
            dimension_semantics=("parallel",),
        ),
    )(flat, emb, x2, y2, iota_row, k)


def _sc_gather(emb, ind):
    info = plsc.get_sparse_core_info()
    nc, ns = info.num_cores, info.num_subcores
    nw = nc * ns
    b = ind.shape[0]
    d = emb.shape[1]
    bpw = b // nw
    mesh = plsc.VectorSubcoreMesh(core_axis_name="c", subcore_axis_name="s")

    @functools.partial(
        pl.kernel,
        mesh=mesh,
        out_type=jax.ShapeDtypeStruct((b, d), jnp.float32),
        scratch_types=[
            pltpu.VMEM((bpw,), jnp.int32),
            pltpu.VMEM((bpw, d), jnp.float32),
            pltpu.SemaphoreType.DMA,
        ],
    )
    def gk(table_hbm, idx_hbm, out_hbm, idx_v, rows_v, sem):
        wid = lax.axis_index("s") * nc + lax.axis_index("c")
        base = wid * bpw
        pltpu.sync_copy(idx_hbm.at[pl.ds(base, bpw)], idx_v)
        pltpu.async_copy(table_hbm.at[idx_v], rows_v, sem).wait()
        pltpu.sync_copy(rows_v, out_hbm.at[pl.ds(base, bpw)])

    return gk(emb, ind)


def kernel(x, k, embed):
    b, n, d = x.shape
    kc = embed.shape[1]
    flat = x.reshape(b * n, d)
    emb = embed.reshape(kc, d)
    x2 = jnp.sum(flat ** 2, axis=-1)[:, None]
    y2 = jnp.sum(emb ** 2, axis=-1)[None, :]
    iota_row = jnp.arange(kc, dtype=jnp.float32)[None, :]
    k_arr = jnp.asarray(k, dtype=jnp.int32).reshape(1, 1)
    dist, ind2d = _dist_topk(flat, emb, x2, y2, iota_row, k_arr)
    ind = ind2d[:, 0]
    quant = _sc_gather(emb, ind)
    quantize = quant.reshape(b, n, d)
    embed_ind = ind.reshape(b, n)
    dist_out = dist.reshape(1, b, n, kc)
    return quantize, embed_ind, dist_out
